# Initial kernel scaffold; baseline (speedup 1.0000x reference)
import jax
import jax.numpy as jnp
from jax.experimental import pallas as pl


def kernel(tokenized_text_prototypes, token_embedding_table, ctx_vectors):
    raise NotImplementedError("write your pallas kernel here")


# speedup vs baseline: 1.0120x; 1.0120x over previous
import functools

import jax
import jax.numpy as jnp
from jax import lax
from jax.experimental import pallas as pl
from jax.experimental.pallas import tpu as pltpu
from jax.experimental.pallas import tpu_sc as plsc

VOCAB = 49408
D = 512
N_CLS = 1000
CTX_LEN = 77
M = 16
HDR = M + 1
G = CTX_LEN - HDR
GP = 64
NPAD = 1024
TOKP = 80
L = 16


def _body(tok_hbm, gidx_hbm, pidx_hbm, table_hbm, ctx_hbm,
          emb_hbm, eof_hbm,
          buf, gidx_v, tok_v, eof_v, pidx_v, gsem, psem,
          *, nc, cpw):
    wid = lax.axis_index("s") * nc + lax.axis_index("c")
    base = wid * cpw

    pltpu.sync_copy(ctx_hbm, buf.at[pl.ds(1, M)])
    pltpu.sync_copy(pidx_hbm.at[pl.ds(0, 1)], pidx_v)
    pltpu.async_copy(table_hbm.at[pidx_v], buf.at[pl.ds(0, 1)], psem).wait()

    pltpu.sync_copy(gidx_hbm.at[pl.ds(base, cpw)], gidx_v)
    pltpu.sync_copy(tok_hbm.at[wid], tok_v)

    def cls_body(c, carry):
        cls = base + c

        @pl.when(cls < N_CLS)
        def _():
            pltpu.async_copy(table_hbm.at[gidx_v.at[c]],
                             buf.at[pl.ds(HDR, GP)], gsem).wait()
            pltpu.sync_copy(buf.at[pl.ds(0, CTX_LEN)], emb_hbm.at[cls])

        return carry

    lax.fori_loop(0, cpw, cls_body, 0)

    for g in range(cpw // L):
        def eof_body(j, mb, g=g):
            m, best = mb
            v = tok_v[j, pl.ds(g * L, L)]
            gt = v > m
            best = jnp.where(gt, jnp.full((L,), j, jnp.int32), best)
            m = jnp.maximum(m, v)
            return m, best

        m0 = jnp.full((L,), -1, jnp.int32)
        b0 = jnp.zeros((L,), jnp.int32)
        _, best = lax.fori_loop(0, CTX_LEN, eof_body, (m0, b0))
        eof_v[pl.ds(g * L, L)] = best

    pltpu.sync_copy(eof_v, eof_hbm.at[pl.ds(base, cpw)])


def kernel(tokenized_text_prototypes, token_embedding_table, ctx_vectors):
    tokens = tokenized_text_prototypes.astype(jnp.int32)
    gidx = jnp.pad(tokens[:, HDR:], ((0, NPAD - N_CLS), (0, GP - G)))
    tokp = jnp.pad(tokens, ((0, NPAD - N_CLS), (0, TOKP - CTX_LEN)),
                   constant_values=-1)
    pidx = jnp.full((8,), tokens[0, 0], jnp.int32)

    info = plsc.get_sparse_core_info()
    nc, ns = info.num_cores, info.num_subcores
    nw = nc * ns
    cpw = NPAD // nw
    tokt = tokp.reshape(nw, cpw, TOKP).transpose(0, 2, 1)

    mesh = plsc.VectorSubcoreMesh(core_axis_name="c", subcore_axis_name="s",
                                  num_cores=nc, num_subcores=ns)
    fn = pl.kernel(
        functools.partial(_body, nc=nc, cpw=cpw),
        out_type=(
            jax.ShapeDtypeStruct((N_CLS, CTX_LEN, D), jnp.float32),
            jax.ShapeDtypeStruct((NPAD,), jnp.int32),
        ),
        mesh=mesh,
        scratch_types=[
            pltpu.VMEM((HDR + GP, D), jnp.float32),
            pltpu.VMEM((cpw, GP), jnp.int32),
            pltpu.VMEM((TOKP, cpw), jnp.int32),
            pltpu.VMEM((cpw,), jnp.int32),
            pltpu.VMEM((1,), jnp.int32),
            pltpu.SemaphoreType.DMA,
            pltpu.SemaphoreType.DMA,
        ],
        compiler_params=pltpu.CompilerParams(use_tc_tiling_on_sc=False),
    )
    emb, eof = fn(tokt, gidx, pidx, token_embedding_table, ctx_vectors)
    return emb, eof[:N_CLS]
